# baseline (speedup 1.0000x reference)
import functools

import jax
import jax.numpy as jnp
from jax.experimental import pallas as pl

S, D = 2048, 2048
H, KV, HD = 16, 4, 128
EPS = 1e-6
SCALE = HD ** -0.5

BQR = 256
BQ = 256
BK = 256
NEG = -1e30


def _rot_half(x):
    half = x.shape[-1] // 2
    return jnp.concatenate([-x[:, half:], x[:, :half]], axis=-1)


def _norm_rope(x, w, c, s):
    var = jnp.mean(x * x, axis=-1, keepdims=True)
    x = x * jax.lax.rsqrt(var + EPS) * w
    return x * c + _rot_half(x) * s


def _qproj_kernel(x_ref, wq_ref, cos_ref, sin_ref, qw_ref, q_ref, g_ref):
    acc = jnp.dot(x_ref[...], wq_ref[...], preferred_element_type=jnp.float32)
    c = cos_ref[...]
    s = sin_ref[...]
    qw = qw_ref[...]
    qs, gs = [], []
    for hh in range(2):
        qh = acc[:, hh * 256:hh * 256 + HD]
        gh = acc[:, hh * 256 + HD:hh * 256 + 2 * HD]
        qs.append(_norm_rope(qh, qw, c, s))
        gs.append(gh)
    q_ref[...] = jnp.concatenate(qs, axis=1)
    g_ref[...] = jnp.concatenate(gs, axis=1)


def _kvproj_kernel(x_ref, wk_ref, wv_ref, cos_ref, sin_ref, kw_ref, k_ref, v_ref):
    xb = x_ref[...]
    kacc = jnp.dot(xb, wk_ref[...], preferred_element_type=jnp.float32)
    v_ref[...] = jnp.dot(xb, wv_ref[...], preferred_element_type=jnp.float32)
    c = cos_ref[...]
    s = sin_ref[...]
    kw = kw_ref[...]
    ks = [_norm_rope(kacc[:, h * HD:(h + 1) * HD], kw, c, s) for h in range(KV)]
    k_ref[...] = jnp.concatenate(ks, axis=1)


def _attn_kernel(q_ref, k_ref, v_ref, g_ref, o_ref):
    i = pl.program_id(1)
    qb = q_ref[...] * SCALE
    rows = jax.lax.broadcasted_iota(jnp.int32, (BQ, BK), 0) + i * BQ

    def body(j, carry):
        m, l, acc = carry
        kb = k_ref[pl.ds(j * BK, BK), :]
        sc = jax.lax.dot_general(qb, kb, (((1,), (1,)), ((), ())),
                                 preferred_element_type=jnp.float32)
        cols = jax.lax.broadcasted_iota(jnp.int32, (BQ, BK), 1) + j * BK
        sc = jnp.where(cols <= rows, sc, NEG)
        m_new = jnp.maximum(m, jnp.max(sc, axis=1, keepdims=True))
        p = jnp.exp(sc - m_new)
        corr = jnp.exp(m - m_new)
        l = l * corr + jnp.sum(p, axis=1, keepdims=True)
        vb = v_ref[pl.ds(j * BK, BK), :]
        acc = acc * corr + jnp.dot(p, vb, preferred_element_type=jnp.float32)
        return m_new, l, acc

    m0 = jnp.full((BQ, 1), NEG, jnp.float32)
    l0 = jnp.zeros((BQ, 1), jnp.float32)
    a0 = jnp.zeros((BQ, HD), jnp.float32)
    nkb = (i + 1) * (BQ // BK)
    m, l, acc = jax.lax.fori_loop(0, nkb, body, (m0, l0, a0))
    o_ref[...] = acc / l * jax.nn.sigmoid(g_ref[...])


def _oproj_kernel(a_ref, wo_ref, y_ref):
    y_ref[...] = jnp.dot(a_ref[...], wo_ref[...], preferred_element_type=jnp.float32)


@functools.partial(jax.jit, static_argnums=())
def kernel(hidden_states, cos, sin, Wq, Wk, Wv, Wo, q_norm_w, k_norm_w):
    x = hidden_states[0]
    c2 = cos[0]
    s2 = sin[0]
    qw = q_norm_w.reshape(1, HD)
    kw = k_norm_w.reshape(1, HD)

    n_i = S // BQR

    q, gate = pl.pallas_call(
        _qproj_kernel,
        grid=(H // 2, n_i),
        in_specs=[
            pl.BlockSpec((BQR, D), lambda n, i: (i, 0)),
            pl.BlockSpec((D, 512), lambda n, i: (0, n)),
            pl.BlockSpec((BQR, HD), lambda n, i: (i, 0)),
            pl.BlockSpec((BQR, HD), lambda n, i: (i, 0)),
            pl.BlockSpec((1, HD), lambda n, i: (0, 0)),
        ],
        out_specs=[
            pl.BlockSpec((BQR, 2 * HD), lambda n, i: (i, n)),
            pl.BlockSpec((BQR, 2 * HD), lambda n, i: (i, n)),
        ],
        out_shape=[
            jax.ShapeDtypeStruct((S, H * HD), jnp.float32),
            jax.ShapeDtypeStruct((S, H * HD), jnp.float32),
        ],
    )(x, Wq, c2, s2, qw)

    k, v = pl.pallas_call(
        _kvproj_kernel,
        grid=(n_i,),
        in_specs=[
            pl.BlockSpec((BQR, D), lambda i: (i, 0)),
            pl.BlockSpec((D, KV * HD), lambda i: (0, 0)),
            pl.BlockSpec((D, KV * HD), lambda i: (0, 0)),
            pl.BlockSpec((BQR, HD), lambda i: (i, 0)),
            pl.BlockSpec((BQR, HD), lambda i: (i, 0)),
            pl.BlockSpec((1, HD), lambda i: (0, 0)),
        ],
        out_specs=[
            pl.BlockSpec((BQR, KV * HD), lambda i: (i, 0)),
            pl.BlockSpec((BQR, KV * HD), lambda i: (i, 0)),
        ],
        out_shape=[
            jax.ShapeDtypeStruct((S, KV * HD), jnp.float32),
            jax.ShapeDtypeStruct((S, KV * HD), jnp.float32),
        ],
    )(x, Wk, Wv, c2, s2, kw)

    n_rep = H // KV
    attn = pl.pallas_call(
        _attn_kernel,
        grid=(H, S // BQ),
        in_specs=[
            pl.BlockSpec((BQ, HD), lambda h, i: (i, h)),
            pl.BlockSpec((S, HD), lambda h, i: (0, h // n_rep)),
            pl.BlockSpec((S, HD), lambda h, i: (0, h // n_rep)),
            pl.BlockSpec((BQ, HD), lambda h, i: (i, h)),
        ],
        out_specs=pl.BlockSpec((BQ, HD), lambda h, i: (i, h)),
        out_shape=jax.ShapeDtypeStruct((S, H * HD), jnp.float32),
    )(q, k, v, gate)

    y = pl.pallas_call(
        _oproj_kernel,
        grid=(D // 512, n_i),
        in_specs=[
            pl.BlockSpec((BQR, H * HD), lambda n, i: (i, 0)),
            pl.BlockSpec((H * HD, 512), lambda n, i: (0, n)),
        ],
        out_specs=pl.BlockSpec((BQR, 512), lambda n, i: (i, n)),
        out_shape=jax.ShapeDtypeStruct((S, D), jnp.float32),
    )(attn, Wo)

    return y[None]
